# baseline (speedup 1.0000x reference)
import functools

import jax
import jax.numpy as jnp
from jax import lax
from jax.experimental import pallas as pl
from jax.experimental.pallas import tpu as pltpu
from jax.experimental.pallas import tpu_sc as plsc

_N = 131072
_PLANE = 512
_F = 16
_LEVELS = 8

_NC = 2
_NS = 16
_NW = _NC * _NS
_PT = _N // _NW
_C = 256
_TAPS = 24

_LEVEL_OFFS = []
_off = 0
for _l in range(_LEVELS):
    _LEVEL_OFFS.append(_off)
    _off += (_PLANE >> _l) ** 2
_PLANE_ROWS = _off


_L0 = _PLANE * _PLANE
_CH = 32


def _build_chunk(l, p, c, base_out, src_ref, srcbase, idxb, rows4, stag, sem,
                 table, wb):
    w = _PLANE >> l
    lg = 9 - l
    n_out = (w * w) // 2
    outbase = p * _PLANE_ROWS + _LEVEL_OFFS[l] + c * n_out

    iota = lax.iota(jnp.int32, 16)

    def idx_body(q, carry):
        j = base_out + q * 16 + iota
        j = jnp.minimum(j, n_out - 1)
        y = (j >> lg) + c * (w // 2)
        x = j & (w - 1)
        s00 = srcbase + (y << (lg + 2)) + (x << 1)
        s10 = s00 + 2 * w
        q64 = q * 64
        idxb[pl.ds(q64, 16)] = s00
        idxb[pl.ds(q64 + 16, 16)] = s00 + 1
        idxb[pl.ds(q64 + 32, 16)] = s10
        idxb[pl.ds(q64 + 48, 16)] = s10 + 1
        return carry

    lax.fori_loop(0, _CH, idx_body, 0, unroll=4)
    pltpu.async_copy(src_ref.at[idxb], rows4, sem).wait()

    def sum_body(o, carry):
        r = lax.shift_right_logical(o, 4) * 64 + (o & 15)
        acc = (rows4[r, :] + rows4[r + 16, :]) + (rows4[r + 32, :] +
                                                  rows4[r + 48, :])
        stag[o, :] = acc * 0.25
        return carry

    lax.fori_loop(0, 16 * _CH, sum_body, 0, unroll=8)
    pltpu.sync_copy(stag.at[pl.ds(0, wb), :],
                    table.at[pl.ds(outbase + base_out, wb), :])


def _build_body(fmf, table, idxb, rows4, stag, sem):
    c = lax.axis_index("c")
    t = lax.axis_index("s")
    w32 = t * _NC + c

    share = _L0 // _NW

    def l0_copy(p, carry):
        pltpu.sync_copy(fmf.at[pl.ds(p * _L0 + w32 * share, share), :],
                        table.at[pl.ds(p * _PLANE_ROWS + w32 * share,
                                       share), :])
        return carry

    lax.fori_loop(0, 3, l0_copy, 0, unroll=False)

    for l in range(1, _LEVELS):
        w = _PLANE >> l
        n_out = (w * w) // 2
        chunks = max(1, n_out // (16 * _CH))
        wb = min(16 * _CH, n_out)

        def plane_body(p, carry, l=l, chunks=chunks, wb=wb):
            if l == 1:
                src_ref, srcbase = fmf, p * _L0
            else:
                src_ref, srcbase = table, p * _PLANE_ROWS + _LEVEL_OFFS[l - 1]
            if chunks >= _NS:
                cpt = chunks // _NS

                def chunk_body(k, carry_k, p=p, src_ref=src_ref,
                               srcbase=srcbase, cpt=cpt):
                    _build_chunk(l, p, c, (t * cpt + k) * 16 * _CH, src_ref,
                                 srcbase, idxb, rows4, stag, sem, table, wb)
                    return carry_k

                lax.fori_loop(0, cpt, chunk_body, 0, unroll=False)
            else:
                @pl.when(t < chunks)
                def _():
                    _build_chunk(l, p, c, t * 16 * _CH, src_ref, srcbase,
                                 idxb, rows4, stag, sem, table, wb)
            return carry

        lax.fori_loop(0, 3, plane_body, 0, unroll=False)
        if l < _LEVELS - 1:
            plsc.subcore_barrier()


def _level_offset(l):
    t = lax.shift_right_logical(jnp.full((16,), 262144, jnp.int32), 2 * l)
    m4 = lax.shift_left(262144 - t, 2)
    return (m4.astype(jnp.float32) / 3.0).astype(jnp.int32)


def _sc_body(table, x0h, x1h, x2h, lvh, out, x0v, x1v, x2v, lvv,
             idxv, wv, rows, outc, sem):
    wid = lax.axis_index("s") * _NC + lax.axis_index("c")
    pt_base = wid * _PT

    def chunk_body(k, carry):
        base = pt_base + k * _C
        pltpu.sync_copy(x0h.at[pl.ds(base, _C)], x0v)
        pltpu.sync_copy(x1h.at[pl.ds(base, _C)], x1v)
        pltpu.sync_copy(x2h.at[pl.ds(base, _C)], x2v)
        pltpu.sync_copy(lvh.at[pl.ds(base, _C)], lvv)

        def stage_a(g, carry_a):
            s16 = g * 16
            xs = (x0v[pl.ds(s16, 16)], x1v[pl.ds(s16, 16)],
                  x2v[pl.ds(s16, 16)])
            lv = lvv[pl.ds(s16, 16)]
            lv = jnp.minimum(jnp.maximum(lv, 0.0), float(_LEVELS - 1))
            l0 = lv.astype(jnp.int32)
            fr = lv - l0.astype(jnp.float32)
            l1 = jnp.minimum(l0 + 1, _LEVELS - 1)
            wmip = (1.0 - fr, fr)
            lidx = (l0, l1)
            offs = tuple(_level_offset(l) for l in lidx)
            wi = tuple(
                lax.shift_right_logical(jnp.full((16,), _PLANE, jnp.int32), l)
                for l in lidx)
            wf = tuple(w.astype(jnp.float32) for w in wi)
            wm1 = tuple(w - 1 for w in wi)

            for p, (a, b) in enumerate(((1, 2), (0, 2), (0, 1))):
                for s in range(2):
                    u = xs[a] * wf[s] - 0.5
                    v = xs[b] * wf[s] - 0.5
                    iu = u.astype(jnp.int32)
                    iuf = iu.astype(jnp.float32)
                    cu = u < iuf
                    iu = jnp.where(cu, iu - 1, iu)
                    fx = u - jnp.where(cu, iuf - 1.0, iuf)
                    iv = v.astype(jnp.int32)
                    ivf = iv.astype(jnp.float32)
                    cv = v < ivf
                    iv = jnp.where(cv, iv - 1, iv)
                    fy = v - jnp.where(cv, ivf - 1.0, ivf)
                    xa = jnp.minimum(jnp.maximum(iu, 0), wm1[s])
                    xb = jnp.minimum(iu + 1, wm1[s])
                    ya = jnp.minimum(jnp.maximum(iv, 0), wm1[s])
                    yb = jnp.minimum(iv + 1, wm1[s])
                    rbase = offs[s] + p * _PLANE_ROWS
                    r0 = rbase + ya * wi[s]
                    r1 = rbase + yb * wi[s]
                    a0 = (1.0 - fy) * wmip[s]
                    a1 = fy * wmip[s]
                    gx1 = fx
                    gx0 = 1.0 - fx
                    t = (p * 8 + s * 4) * _C + s16
                    idxv[pl.ds(t + 0 * _C, 16)] = r0 + xa
                    idxv[pl.ds(t + 1 * _C, 16)] = r0 + xb
                    idxv[pl.ds(t + 2 * _C, 16)] = r1 + xa
                    idxv[pl.ds(t + 3 * _C, 16)] = r1 + xb
                    wv[pl.ds(t + 0 * _C, 16)] = gx0 * a0
                    wv[pl.ds(t + 1 * _C, 16)] = gx1 * a0
                    wv[pl.ds(t + 2 * _C, 16)] = gx0 * a1
                    wv[pl.ds(t + 3 * _C, 16)] = gx1 * a1
            return carry_a

        lax.fori_loop(0, _C // 16, stage_a, 0, unroll=False)

        pltpu.async_copy(table.at[idxv], rows, sem).wait()

        def stage_c(g, carry_c):
            s16 = g * 16
            wvecs = [wv[pl.ds(t * _C + s16, 16)] for t in range(_TAPS)]
            for ii in range(16):
                i = s16 + ii
                for p in range(3):
                    t0 = p * 8
                    acc = rows[t0 * _C + i, :] * wvecs[t0][ii]
                    for t in range(t0 + 1, t0 + 8):
                        acc = acc + rows[t * _C + i, :] * wvecs[t][ii]
                    outc[i, pl.ds(p * 16, 16)] = acc
            return carry_c

        lax.fori_loop(0, _C // 16, stage_c, 0, unroll=False)

        pltpu.sync_copy(outc, out.at[pl.ds(base, _C), :])
        return carry

    lax.fori_loop(0, _PT // _C, chunk_body, 0, unroll=False)


@jax.jit
def kernel(x, level, fm):
    mesh_b = plsc.VectorSubcoreMesh(core_axis_name="c", subcore_axis_name="s",
                                    num_cores=_NC, num_subcores=_NS)
    build = pl.kernel(
        _build_body,
        out_type=jax.ShapeDtypeStruct((3 * _PLANE_ROWS, _F), jnp.float32),
        mesh=mesh_b,
        scratch_types=[
            pltpu.VMEM((4 * 16 * _CH,), jnp.int32),
            pltpu.VMEM((4 * 16 * _CH, _F), jnp.float32),
            pltpu.VMEM((16 * _CH, _F), jnp.float32),
            pltpu.SemaphoreType.DMA,
        ],
        compiler_params=pltpu.CompilerParams(use_tc_tiling_on_sc=False),
        name="scbuild",
    )
    table = build(fm.reshape(3 * _L0, _F))

    x0 = x[:, 0]
    x1 = x[:, 1]
    x2 = x[:, 2]
    lv = level[:, 0]

    mesh = plsc.VectorSubcoreMesh(core_axis_name="c", subcore_axis_name="s",
                                  num_cores=_NC, num_subcores=_NS)
    sc = pl.kernel(
        _sc_body,
        out_type=jax.ShapeDtypeStruct((_N, 3 * _F), jnp.float32),
        mesh=mesh,
        scratch_types=[
            pltpu.VMEM((_C,), jnp.float32),
            pltpu.VMEM((_C,), jnp.float32),
            pltpu.VMEM((_C,), jnp.float32),
            pltpu.VMEM((_C,), jnp.float32),
            pltpu.VMEM((_TAPS * _C,), jnp.int32),
            pltpu.VMEM((_TAPS * _C,), jnp.float32),
            pltpu.VMEM((_TAPS * _C, _F), jnp.float32),
            pltpu.VMEM((_C, 3 * _F), jnp.float32),
            pltpu.SemaphoreType.DMA,
        ],
        compiler_params=pltpu.CompilerParams(use_tc_tiling_on_sc=False),
        name="scsample",
    )
    return sc(table, x0, x1, x2, lv)


# speedup vs baseline: 2.3170x; 2.3170x over previous
import functools

import jax
import jax.numpy as jnp
from jax import lax
from jax.experimental import pallas as pl
from jax.experimental.pallas import tpu as pltpu
from jax.experimental.pallas import tpu_sc as plsc

_N = 131072
_PLANE = 512
_F = 16
_LEVELS = 8

_NC = 2
_NS = 16
_NW = _NC * _NS
_PT = _N // _NW
_C = 256
_TAPS = 24

_LEVEL_OFFS = []
_off = 0
for _l in range(_LEVELS):
    _LEVEL_OFFS.append(_off)
    _off += (_PLANE >> _l) ** 2
_PLANE_ROWS = _off


_L0 = _PLANE * _PLANE
_CH = 32


def _build_chunk(l, p, c, base_out, src_ref, srcbase, idxb, rows4, stag, sem,
                 table, wb):
    w = _PLANE >> l
    lg = 9 - l
    n_out = (w * w) // 2
    outbase = p * _PLANE_ROWS + _LEVEL_OFFS[l] + c * n_out

    iota = lax.iota(jnp.int32, 16)

    def idx_body(q, carry):
        j = base_out + q * 16 + iota
        j = jnp.minimum(j, n_out - 1)
        y = (j >> lg) + c * (w // 2)
        x = j & (w - 1)
        s00 = srcbase + (y << (lg + 2)) + (x << 1)
        s10 = s00 + 2 * w
        q64 = q * 64
        idxb[pl.ds(q64, 16)] = s00
        idxb[pl.ds(q64 + 16, 16)] = s00 + 1
        idxb[pl.ds(q64 + 32, 16)] = s10
        idxb[pl.ds(q64 + 48, 16)] = s10 + 1
        return carry

    lax.fori_loop(0, _CH, idx_body, 0, unroll=4)
    pltpu.async_copy(src_ref.at[idxb], rows4, sem).wait()

    def sum_body(o, carry):
        r = lax.shift_right_logical(o, 4) * 64 + (o & 15)
        acc = (rows4[r, :] + rows4[r + 16, :]) + (rows4[r + 32, :] +
                                                  rows4[r + 48, :])
        stag[o, :] = acc * 0.25
        return carry

    lax.fori_loop(0, 16 * _CH, sum_body, 0, unroll=8)
    pltpu.sync_copy(stag.at[pl.ds(0, wb), :],
                    table.at[pl.ds(outbase + base_out, wb), :])


def _build_body(fmf, table, idxb, rows4, stag, sem):
    c = lax.axis_index("c")
    t = lax.axis_index("s")
    w32 = t * _NC + c

    share = _L0 // _NW

    nstage = 4 * 16 * _CH
    nhops = share // nstage

    def l0_copy(i, carry):
        p = i // nhops
        k = i - p * nhops
        src = p * _L0 + w32 * share + k * nstage
        dst = p * _PLANE_ROWS + w32 * share + k * nstage
        pltpu.sync_copy(fmf.at[pl.ds(src, nstage), :], rows4)
        pltpu.sync_copy(rows4, table.at[pl.ds(dst, nstage), :])
        return carry

    lax.fori_loop(0, 3 * nhops, l0_copy, 0, unroll=False)

    for l in range(1, _LEVELS):
        w = _PLANE >> l
        n_out = (w * w) // 2
        chunks = max(1, n_out // (16 * _CH))
        wb = min(16 * _CH, n_out)

        def plane_body(p, carry, l=l, chunks=chunks, wb=wb):
            if l == 1:
                src_ref, srcbase = fmf, p * _L0
            else:
                src_ref, srcbase = table, p * _PLANE_ROWS + _LEVEL_OFFS[l - 1]
            if chunks >= _NS:
                cpt = chunks // _NS

                def chunk_body(k, carry_k, p=p, src_ref=src_ref,
                               srcbase=srcbase, cpt=cpt):
                    _build_chunk(l, p, c, (t * cpt + k) * 16 * _CH, src_ref,
                                 srcbase, idxb, rows4, stag, sem, table, wb)
                    return carry_k

                lax.fori_loop(0, cpt, chunk_body, 0, unroll=False)
            else:
                @pl.when(t < chunks)
                def _():
                    _build_chunk(l, p, c, t * 16 * _CH, src_ref, srcbase,
                                 idxb, rows4, stag, sem, table, wb)
            return carry

        lax.fori_loop(0, 3, plane_body, 0, unroll=False)
        if l < _LEVELS - 1:
            plsc.subcore_barrier()


def _level_offset(l):
    t = lax.shift_right_logical(jnp.full((16,), 262144, jnp.int32), 2 * l)
    m4 = lax.shift_left(262144 - t, 2)
    return (m4.astype(jnp.float32) / 3.0).astype(jnp.int32)


def _sc_body(table, x0h, x1h, x2h, lvh, out, x0v, x1v, x2v, lvv,
             idxv, wv, rows, outc, sem):
    wid = lax.axis_index("s") * _NC + lax.axis_index("c")
    pt_base = wid * _PT

    def chunk_body(k, carry):
        base = pt_base + k * _C
        pltpu.sync_copy(x0h.at[pl.ds(base, _C)], x0v)
        pltpu.sync_copy(x1h.at[pl.ds(base, _C)], x1v)
        pltpu.sync_copy(x2h.at[pl.ds(base, _C)], x2v)
        pltpu.sync_copy(lvh.at[pl.ds(base, _C)], lvv)

        def stage_a(g, carry_a):
            s16 = g * 16
            xs = (x0v[pl.ds(s16, 16)], x1v[pl.ds(s16, 16)],
                  x2v[pl.ds(s16, 16)])
            lv = lvv[pl.ds(s16, 16)]
            lv = jnp.minimum(jnp.maximum(lv, 0.0), float(_LEVELS - 1))
            l0 = lv.astype(jnp.int32)
            fr = lv - l0.astype(jnp.float32)
            l1 = jnp.minimum(l0 + 1, _LEVELS - 1)
            wmip = (1.0 - fr, fr)
            lidx = (l0, l1)
            offs = tuple(_level_offset(l) for l in lidx)
            wi = tuple(
                lax.shift_right_logical(jnp.full((16,), _PLANE, jnp.int32), l)
                for l in lidx)
            wf = tuple(w.astype(jnp.float32) for w in wi)
            wm1 = tuple(w - 1 for w in wi)

            for p, (a, b) in enumerate(((1, 2), (0, 2), (0, 1))):
                for s in range(2):
                    u = xs[a] * wf[s] - 0.5
                    v = xs[b] * wf[s] - 0.5
                    iu = u.astype(jnp.int32)
                    iuf = iu.astype(jnp.float32)
                    cu = u < iuf
                    iu = jnp.where(cu, iu - 1, iu)
                    fx = u - jnp.where(cu, iuf - 1.0, iuf)
                    iv = v.astype(jnp.int32)
                    ivf = iv.astype(jnp.float32)
                    cv = v < ivf
                    iv = jnp.where(cv, iv - 1, iv)
                    fy = v - jnp.where(cv, ivf - 1.0, ivf)
                    xa = jnp.minimum(jnp.maximum(iu, 0), wm1[s])
                    xb = jnp.minimum(iu + 1, wm1[s])
                    ya = jnp.minimum(jnp.maximum(iv, 0), wm1[s])
                    yb = jnp.minimum(iv + 1, wm1[s])
                    rbase = offs[s] + p * _PLANE_ROWS
                    r0 = rbase + ya * wi[s]
                    r1 = rbase + yb * wi[s]
                    a0 = (1.0 - fy) * wmip[s]
                    a1 = fy * wmip[s]
                    gx1 = fx
                    gx0 = 1.0 - fx
                    t = (p * 8 + s * 4) * _C + s16
                    idxv[pl.ds(t + 0 * _C, 16)] = r0 + xa
                    idxv[pl.ds(t + 1 * _C, 16)] = r0 + xb
                    idxv[pl.ds(t + 2 * _C, 16)] = r1 + xa
                    idxv[pl.ds(t + 3 * _C, 16)] = r1 + xb
                    wv[pl.ds(t + 0 * _C, 16)] = gx0 * a0
                    wv[pl.ds(t + 1 * _C, 16)] = gx1 * a0
                    wv[pl.ds(t + 2 * _C, 16)] = gx0 * a1
                    wv[pl.ds(t + 3 * _C, 16)] = gx1 * a1
            return carry_a

        lax.fori_loop(0, _C // 16, stage_a, 0, unroll=False)

        pltpu.async_copy(table.at[idxv], rows, sem).wait()

        def stage_c(g, carry_c):
            s16 = g * 16
            wvecs = [wv[pl.ds(t * _C + s16, 16)] for t in range(_TAPS)]
            for ii in range(16):
                i = s16 + ii
                for p in range(3):
                    t0 = p * 8
                    acc = rows[t0 * _C + i, :] * wvecs[t0][ii]
                    for t in range(t0 + 1, t0 + 8):
                        acc = acc + rows[t * _C + i, :] * wvecs[t][ii]
                    outc[i, pl.ds(p * 16, 16)] = acc
            return carry_c

        lax.fori_loop(0, _C // 16, stage_c, 0, unroll=False)

        pltpu.sync_copy(outc, out.at[pl.ds(base, _C), :])
        return carry

    lax.fori_loop(0, _PT // _C, chunk_body, 0, unroll=False)


@jax.jit
def kernel(x, level, fm):
    mesh_b = plsc.VectorSubcoreMesh(core_axis_name="c", subcore_axis_name="s",
                                    num_cores=_NC, num_subcores=_NS)
    build = pl.kernel(
        _build_body,
        out_type=jax.ShapeDtypeStruct((3 * _PLANE_ROWS, _F), jnp.float32),
        mesh=mesh_b,
        scratch_types=[
            pltpu.VMEM((4 * 16 * _CH,), jnp.int32),
            pltpu.VMEM((4 * 16 * _CH, _F), jnp.float32),
            pltpu.VMEM((16 * _CH, _F), jnp.float32),
            pltpu.SemaphoreType.DMA,
        ],
        compiler_params=pltpu.CompilerParams(use_tc_tiling_on_sc=False),
        name="scbuild",
    )
    table = build(fm.reshape(3 * _L0, _F))

    x0 = x[:, 0]
    x1 = x[:, 1]
    x2 = x[:, 2]
    lv = level[:, 0]

    mesh = plsc.VectorSubcoreMesh(core_axis_name="c", subcore_axis_name="s",
                                  num_cores=_NC, num_subcores=_NS)
    sc = pl.kernel(
        _sc_body,
        out_type=jax.ShapeDtypeStruct((_N, 3 * _F), jnp.float32),
        mesh=mesh,
        scratch_types=[
            pltpu.VMEM((_C,), jnp.float32),
            pltpu.VMEM((_C,), jnp.float32),
            pltpu.VMEM((_C,), jnp.float32),
            pltpu.VMEM((_C,), jnp.float32),
            pltpu.VMEM((_TAPS * _C,), jnp.int32),
            pltpu.VMEM((_TAPS * _C,), jnp.float32),
            pltpu.VMEM((_TAPS * _C, _F), jnp.float32),
            pltpu.VMEM((_C, 3 * _F), jnp.float32),
            pltpu.SemaphoreType.DMA,
        ],
        compiler_params=pltpu.CompilerParams(use_tc_tiling_on_sc=False),
        name="scsample",
    )
    return sc(table, x0, x1, x2, lv)


# speedup vs baseline: 2.6204x; 1.1309x over previous
import functools

import jax
import jax.numpy as jnp
from jax import lax
from jax.experimental import pallas as pl
from jax.experimental.pallas import tpu as pltpu
from jax.experimental.pallas import tpu_sc as plsc

_N = 131072
_PLANE = 512
_F = 16
_LEVELS = 8

_NC = 2
_NS = 16
_NW = _NC * _NS
_PT = _N // _NW
_C = 256
_TAPS = 24

_LEVEL_OFFS = []
_off = 0
for _l in range(_LEVELS):
    _LEVEL_OFFS.append(_off)
    _off += (_PLANE >> _l) ** 2
_PLANE_ROWS = _off


_L0 = _PLANE * _PLANE
_CH = 32


def _build_chunk(l, p, c, base_out, src_ref, srcbase, idxb, rows4, stag, sem,
                 table, wb):
    w = _PLANE >> l
    lg = 9 - l
    n_out = (w * w) // 2
    outbase = p * _PLANE_ROWS + _LEVEL_OFFS[l] + c * n_out

    y0 = (base_out >> lg) + c * (w // 2)
    slab0 = srcbase + (y0 << (lg + 2))
    pltpu.sync_copy(src_ref.at[pl.ds(slab0, 4 * wb), :],
                    rows4.at[pl.ds(0, 4 * wb), :])

    def sum_body(o, carry):
        yy = o >> lg
        x = o & (w - 1)
        r0 = (yy << (lg + 2)) + (x << 1)
        r1 = r0 + 2 * w
        acc = (rows4[r0, :] + rows4[r0 + 1, :]) + (rows4[r1, :] +
                                                   rows4[r1 + 1, :])
        stag[o, :] = acc * 0.25
        return carry

    lax.fori_loop(0, wb, sum_body, 0, unroll=8)
    pltpu.sync_copy(stag.at[pl.ds(0, wb), :],
                    table.at[pl.ds(outbase + base_out, wb), :])


def _build_body(fmf, table, idxb, rows4, stag, sem):
    c = lax.axis_index("c")
    t = lax.axis_index("s")
    w32 = t * _NC + c

    share = _L0 // _NW

    nstage = 4 * 16 * _CH
    nhops = share // nstage

    def l0_copy(i, carry):
        p = i // nhops
        k = i - p * nhops
        src = p * _L0 + w32 * share + k * nstage
        dst = p * _PLANE_ROWS + w32 * share + k * nstage
        pltpu.sync_copy(fmf.at[pl.ds(src, nstage), :], rows4)
        pltpu.sync_copy(rows4, table.at[pl.ds(dst, nstage), :])
        return carry

    lax.fori_loop(0, 3 * nhops, l0_copy, 0, unroll=False)

    for l in range(1, _LEVELS):
        w = _PLANE >> l
        n_out = (w * w) // 2
        chunks = max(1, n_out // (16 * _CH))
        wb = min(16 * _CH, n_out)

        def plane_body(p, carry, l=l, chunks=chunks, wb=wb):
            if l == 1:
                src_ref, srcbase = fmf, p * _L0
            else:
                src_ref, srcbase = table, p * _PLANE_ROWS + _LEVEL_OFFS[l - 1]
            if chunks >= _NS:
                cpt = chunks // _NS

                def chunk_body(k, carry_k, p=p, src_ref=src_ref,
                               srcbase=srcbase, cpt=cpt):
                    _build_chunk(l, p, c, (t * cpt + k) * 16 * _CH, src_ref,
                                 srcbase, idxb, rows4, stag, sem, table, wb)
                    return carry_k

                lax.fori_loop(0, cpt, chunk_body, 0, unroll=False)
            else:
                @pl.when(t < chunks)
                def _():
                    _build_chunk(l, p, c, t * 16 * _CH, src_ref, srcbase,
                                 idxb, rows4, stag, sem, table, wb)
            return carry

        lax.fori_loop(0, 3, plane_body, 0, unroll=False)
        if l < _LEVELS - 1:
            plsc.subcore_barrier()


def _level_offset(l):
    t = lax.shift_right_logical(jnp.full((16,), 262144, jnp.int32), 2 * l)
    m4 = lax.shift_left(262144 - t, 2)
    return (m4.astype(jnp.float32) / 3.0).astype(jnp.int32)


def _sc_body(table, x0h, x1h, x2h, lvh, out, x0v, x1v, x2v, lvv,
             idxv, wv, rows, outc, sem):
    wid = lax.axis_index("s") * _NC + lax.axis_index("c")
    pt_base = wid * _PT

    def chunk_body(k, carry):
        base = pt_base + k * _C
        pltpu.sync_copy(x0h.at[pl.ds(base, _C)], x0v)
        pltpu.sync_copy(x1h.at[pl.ds(base, _C)], x1v)
        pltpu.sync_copy(x2h.at[pl.ds(base, _C)], x2v)
        pltpu.sync_copy(lvh.at[pl.ds(base, _C)], lvv)

        def stage_a(g, carry_a):
            s16 = g * 16
            xs = (x0v[pl.ds(s16, 16)], x1v[pl.ds(s16, 16)],
                  x2v[pl.ds(s16, 16)])
            lv = lvv[pl.ds(s16, 16)]
            lv = jnp.minimum(jnp.maximum(lv, 0.0), float(_LEVELS - 1))
            l0 = lv.astype(jnp.int32)
            fr = lv - l0.astype(jnp.float32)
            l1 = jnp.minimum(l0 + 1, _LEVELS - 1)
            wmip = (1.0 - fr, fr)
            lidx = (l0, l1)
            offs = tuple(_level_offset(l) for l in lidx)
            wi = tuple(
                lax.shift_right_logical(jnp.full((16,), _PLANE, jnp.int32), l)
                for l in lidx)
            wf = tuple(w.astype(jnp.float32) for w in wi)
            wm1 = tuple(w - 1 for w in wi)

            for p, (a, b) in enumerate(((1, 2), (0, 2), (0, 1))):
                for s in range(2):
                    u = xs[a] * wf[s] - 0.5
                    v = xs[b] * wf[s] - 0.5
                    iu = u.astype(jnp.int32)
                    iuf = iu.astype(jnp.float32)
                    cu = u < iuf
                    iu = jnp.where(cu, iu - 1, iu)
                    fx = u - jnp.where(cu, iuf - 1.0, iuf)
                    iv = v.astype(jnp.int32)
                    ivf = iv.astype(jnp.float32)
                    cv = v < ivf
                    iv = jnp.where(cv, iv - 1, iv)
                    fy = v - jnp.where(cv, ivf - 1.0, ivf)
                    xa = jnp.minimum(jnp.maximum(iu, 0), wm1[s])
                    xb = jnp.minimum(iu + 1, wm1[s])
                    ya = jnp.minimum(jnp.maximum(iv, 0), wm1[s])
                    yb = jnp.minimum(iv + 1, wm1[s])
                    rbase = offs[s] + p * _PLANE_ROWS
                    r0 = rbase + ya * wi[s]
                    r1 = rbase + yb * wi[s]
                    a0 = (1.0 - fy) * wmip[s]
                    a1 = fy * wmip[s]
                    gx1 = fx
                    gx0 = 1.0 - fx
                    t = (p * 8 + s * 4) * _C + s16
                    idxv[pl.ds(t + 0 * _C, 16)] = r0 + xa
                    idxv[pl.ds(t + 1 * _C, 16)] = r0 + xb
                    idxv[pl.ds(t + 2 * _C, 16)] = r1 + xa
                    idxv[pl.ds(t + 3 * _C, 16)] = r1 + xb
                    wv[pl.ds(t + 0 * _C, 16)] = gx0 * a0
                    wv[pl.ds(t + 1 * _C, 16)] = gx1 * a0
                    wv[pl.ds(t + 2 * _C, 16)] = gx0 * a1
                    wv[pl.ds(t + 3 * _C, 16)] = gx1 * a1
            return carry_a

        lax.fori_loop(0, _C // 16, stage_a, 0, unroll=False)

        pltpu.async_copy(table.at[idxv], rows, sem).wait()

        def stage_c(g, carry_c):
            s16 = g * 16
            wvecs = [wv[pl.ds(t * _C + s16, 16)] for t in range(_TAPS)]
            for ii in range(16):
                i = s16 + ii
                for p in range(3):
                    t0 = p * 8
                    acc = rows[t0 * _C + i, :] * wvecs[t0][ii]
                    for t in range(t0 + 1, t0 + 8):
                        acc = acc + rows[t * _C + i, :] * wvecs[t][ii]
                    outc[i, pl.ds(p * 16, 16)] = acc
            return carry_c

        lax.fori_loop(0, _C // 16, stage_c, 0, unroll=False)

        pltpu.sync_copy(outc, out.at[pl.ds(base, _C), :])
        return carry

    lax.fori_loop(0, _PT // _C, chunk_body, 0, unroll=False)


@jax.jit
def kernel(x, level, fm):
    mesh_b = plsc.VectorSubcoreMesh(core_axis_name="c", subcore_axis_name="s",
                                    num_cores=_NC, num_subcores=_NS)
    build = pl.kernel(
        _build_body,
        out_type=jax.ShapeDtypeStruct((3 * _PLANE_ROWS, _F), jnp.float32),
        mesh=mesh_b,
        scratch_types=[
            pltpu.VMEM((4 * 16 * _CH,), jnp.int32),
            pltpu.VMEM((4 * 16 * _CH, _F), jnp.float32),
            pltpu.VMEM((16 * _CH, _F), jnp.float32),
            pltpu.SemaphoreType.DMA,
        ],
        compiler_params=pltpu.CompilerParams(use_tc_tiling_on_sc=False),
        name="scbuild",
    )
    table = build(fm.reshape(3 * _L0, _F))

    x0 = x[:, 0]
    x1 = x[:, 1]
    x2 = x[:, 2]
    lv = level[:, 0]

    mesh = plsc.VectorSubcoreMesh(core_axis_name="c", subcore_axis_name="s",
                                  num_cores=_NC, num_subcores=_NS)
    sc = pl.kernel(
        _sc_body,
        out_type=jax.ShapeDtypeStruct((_N, 3 * _F), jnp.float32),
        mesh=mesh,
        scratch_types=[
            pltpu.VMEM((_C,), jnp.float32),
            pltpu.VMEM((_C,), jnp.float32),
            pltpu.VMEM((_C,), jnp.float32),
            pltpu.VMEM((_C,), jnp.float32),
            pltpu.VMEM((_TAPS * _C,), jnp.int32),
            pltpu.VMEM((_TAPS * _C,), jnp.float32),
            pltpu.VMEM((_TAPS * _C, _F), jnp.float32),
            pltpu.VMEM((_C, 3 * _F), jnp.float32),
            pltpu.SemaphoreType.DMA,
        ],
        compiler_params=pltpu.CompilerParams(use_tc_tiling_on_sc=False),
        name="scsample",
    )
    return sc(table, x0, x1, x2, lv)
